# baseline (speedup 1.0000x reference)
import functools

import jax
import jax.numpy as jnp
from jax import lax
from jax.experimental import pallas as pl
from jax.experimental.pallas import tpu as pltpu
from jax.experimental.pallas import tpu_sc as plsc

NC = 2
NS = 16
NW = NC * NS

CHUNK = 80
NEG_INF = -3.0e38


def _ka_body(h_ref, wfc_ref, wst_ref, zs_ref, st_ref):
    zb = jnp.dot(h_ref[...], wfc_ref[...], preferred_element_type=jnp.float32)
    hd = zb.shape[1] // 2
    zs_ref[...] = jnp.stack([zb[:, :hd], zb[:, hd:]], axis=0)
    st = jnp.dot(zb, wst_ref[...], preferred_element_type=jnp.float32)
    st_ref[...] = st.T[None]


def _phase_a(h, W_fc, wst, n, d, nb):
    grid = n // nb
    return pl.pallas_call(
        _ka_body,
        grid=(grid,),
        in_specs=[
            pl.BlockSpec((nb, d), lambda i: (i, 0)),
            pl.BlockSpec((d, d), lambda i: (0, 0)),
            pl.BlockSpec((d, 2), lambda i: (0, 0)),
        ],
        out_specs=[
            pl.BlockSpec((2, nb, d // 2), lambda i: (0, i, 0)),
            pl.BlockSpec((1, 2, nb), lambda i: (i, 0, 0)),
        ],
        out_shape=[
            jax.ShapeDtypeStruct((2, n, d // 2), jnp.float32),
            jax.ShapeDtypeStruct((grid, 2, nb), jnp.float32),
        ],
    )(h, W_fc, wst)


def _kb1_body(e_ref, weatt_ref, we2n_ref, w7_ref, ezs_ref, u_ref):
    v2 = jnp.dot(weatt_ref[...], we2n_ref[...], preferred_element_type=jnp.float32)
    v7 = jnp.dot(weatt_ref[...], w7_ref[...], preferred_element_type=jnp.float32)
    eb = e_ref[...]
    ez = jnp.dot(eb, v2, preferred_element_type=jnp.float32)
    hd = ez.shape[1] // 2
    ezs_ref[...] = jnp.stack([ez[:, :hd], ez[:, hd:]], axis=0)
    u = jnp.dot(eb, v7, preferred_element_type=jnp.float32)
    u_ref[...] = u.reshape(1, 1, -1)


def _phase_b1(e, W_eatt, W_e2n, w7, n_e, ed, d, eb):
    grid = n_e // eb
    return pl.pallas_call(
        _kb1_body,
        grid=(grid,),
        in_specs=[
            pl.BlockSpec((eb, ed), lambda i: (i, 0)),
            pl.BlockSpec((ed, ed), lambda i: (0, 0)),
            pl.BlockSpec((ed, d), lambda i: (0, 0)),
            pl.BlockSpec((ed, 1), lambda i: (0, 0)),
        ],
        out_specs=[
            pl.BlockSpec((2, eb, d // 2), lambda i: (0, i, 0)),
            pl.BlockSpec((1, 1, eb), lambda i: (i, 0, 0)),
        ],
        out_shape=[
            jax.ShapeDtypeStruct((2, n_e, d // 2), jnp.float32),
            jax.ShapeDtypeStruct((grid, 1, eb), jnp.float32),
        ],
    )(e, W_eatt, W_e2n, w7)


def _kb2_body(n, epw, st_hbm, u_hbm, src_hbm, dst_hbm,
              b_hbm, bmax_hbm, cnts_hbm, cntd_hbm,
              s_v, t_v, u_v, src_v, dst_v, b_v, cnt_s, cnt_d, bmax_v):
    cid = lax.axis_index("c")
    sid = lax.axis_index("s")
    wid = sid * NC + cid
    base = wid * epw

    pltpu.sync_copy(st_hbm.at[0, 0], s_v)
    pltpu.sync_copy(st_hbm.at[1, 0], t_v)
    pltpu.sync_copy(u_hbm.at[pl.ds(base, epw)], u_v)
    pltpu.sync_copy(src_hbm.at[pl.ds(base, epw)], src_v)
    pltpu.sync_copy(dst_hbm.at[pl.ds(base, epw)], dst_v)

    zeros16 = jnp.zeros((16,), jnp.float32)

    def zero_body(j, _):
        cnt_s[pl.ds(j * 16, 16)] = zeros16
        cnt_d[pl.ds(j * 16, 16)] = zeros16
        return 0

    lax.fori_loop(0, n // 16, zero_body, 0)

    onesf = jnp.ones((16,), jnp.float32)

    def body(j, bmax):
        sl = pl.ds(j * 16, 16)
        src16 = src_v[sl]
        dst16 = dst_v[sl]
        s16 = plsc.load_gather(s_v, [src16])
        t16 = plsc.load_gather(t_v, [dst16])
        b16 = s16 + t16 + u_v[sl]
        b_v[sl] = b16
        plsc.addupdate_scatter(cnt_s, [src16], onesf)
        plsc.addupdate_scatter(cnt_d, [dst16], onesf)
        return jnp.maximum(bmax, b16)

    bmax = lax.fori_loop(0, epw // 16, body, jnp.full((16,), NEG_INF, jnp.float32))
    bmax_v[...] = bmax

    pltpu.sync_copy(b_v, b_hbm.at[pl.ds(base, epw)])
    pltpu.sync_copy(bmax_v, bmax_hbm.at[wid, 0])
    pltpu.sync_copy(cnt_s, cnts_hbm.at[wid, 0])
    pltpu.sync_copy(cnt_d, cntd_hbm.at[wid, 0])


def _phase_b2(st, u, src, dst, n, n_e):
    epw = n_e // NW
    mesh = plsc.VectorSubcoreMesh(core_axis_name="c", subcore_axis_name="s")
    return pl.kernel(
        functools.partial(_kb2_body, n, epw),
        compiler_params=pltpu.CompilerParams(needs_layout_passes=False),
        out_type=[
            jax.ShapeDtypeStruct((n_e,), jnp.float32),
            jax.ShapeDtypeStruct((NW, 1, 16), jnp.float32),
            jax.ShapeDtypeStruct((NW, 1, n), jnp.float32),
            jax.ShapeDtypeStruct((NW, 1, n), jnp.float32),
        ],
        mesh=mesh,
        scratch_types=[
            pltpu.VMEM((n,), jnp.float32),
            pltpu.VMEM((n,), jnp.float32),
            pltpu.VMEM((epw,), jnp.float32),
            pltpu.VMEM((epw,), jnp.int32),
            pltpu.VMEM((epw,), jnp.int32),
            pltpu.VMEM((epw,), jnp.float32),
            pltpu.VMEM((n,), jnp.float32),
            pltpu.VMEM((n,), jnp.float32),
            pltpu.VMEM((16,), jnp.float32),
        ],
    )(st, u, src, dst)


def _kc_body(n_e, cnts_ref, cntd_ref, zs_ref, bmaxp_ref, wm1_ref, wm2_ref,
             wattn_ref, params_ref):
    cnt = jnp.stack([jnp.sum(cnts_ref[...], axis=(0, 1)),
                     jnp.sum(cntd_ref[...], axis=(0, 1))])
    z = jnp.concatenate([zs_ref[0], zs_ref[1]], axis=1)
    inv_e = 1.0 / n_e
    ex1 = jnp.dot(cnt, z, preferred_element_type=jnp.float32) * inv_e
    ex2 = jnp.dot(cnt, z * z, preferred_element_type=jnp.float32) * inv_e
    ex3 = jnp.dot(cnt, z * z * z, preferred_element_type=jnp.float32) * inv_e
    m2 = ex2 - ex1 * ex1
    m3 = ex3 - 3.0 * ex1 * ex2 + 2.0 * ex1 * ex1 * ex1
    sr2 = jnp.sign(m2) * jnp.sqrt(jnp.abs(m2))
    sr3 = jnp.sign(m3) * jnp.abs(m3) ** (1.0 / 3.0)
    a2 = jnp.dot(sr2, wm1_ref[...], preferred_element_type=jnp.float32)
    a3 = jnp.dot(sr3, wm2_ref[...], preferred_element_type=jnp.float32)
    wa = wattn_ref[...][:, 0]
    d = z.shape[1]
    c = (jnp.sum(a2[0] * wa[d:2 * d]) + jnp.sum(a3[0] * wa[2 * d:3 * d])
         + jnp.sum(a2[1] * wa[4 * d:5 * d]) + jnp.sum(a3[1] * wa[5 * d:6 * d]))
    gb = jnp.max(bmaxp_ref[...]) + c
    gmax = jnp.where(gb > 0, gb, 0.01 * gb)
    lane = lax.broadcasted_iota(jnp.int32, (16,), 0)
    params_ref[...] = jnp.where(lane == 0, c, jnp.where(lane == 1, gmax, 0.0))


def _phase_c(cnts, cntd, zs, bmaxp, W_m1, W_m2, W_attn, n, d, n_e):
    return pl.pallas_call(
        functools.partial(_kc_body, float(n_e)),
        grid=(1,),
        in_specs=[
            pl.BlockSpec((NW, 1, n), lambda i: (0, 0, 0)),
            pl.BlockSpec((NW, 1, n), lambda i: (0, 0, 0)),
            pl.BlockSpec((2, n, d // 2), lambda i: (0, 0, 0)),
            pl.BlockSpec((NW, 1, 16), lambda i: (0, 0, 0)),
            pl.BlockSpec((d, d), lambda i: (0, 0)),
            pl.BlockSpec((d, d), lambda i: (0, 0)),
            pl.BlockSpec((6 * d + 16, 1), lambda i: (0, 0)),
        ],
        out_specs=pl.BlockSpec((16,), lambda i: (0,)),
        out_shape=jax.ShapeDtypeStruct((16,), jnp.float32),
    )(cnts, cntd, zs, bmaxp, W_m1, W_m2, W_attn)


def _kd_body(n, n_e, hd, src_hbm, dst_hbm, b_hbm, params_hbm, zs_hbm, ezs_hbm,
             accp_hbm, denp_hbm,
             src_v, dst_v, b_v, params_v, zr0, zr1, ez0, ez1, zbuf_v,
             gs0, gs1, es0, es1, ss0, ss1, dsem,
             acc_sp, den_sp):
    cid = lax.axis_index("c")
    sid = lax.axis_index("s")
    ept = n_e // NS
    base = sid * ept
    nchunks = ept // CHUNK
    out_chunks = n // CHUNK

    pltpu.sync_copy(src_hbm.at[sid], src_v)
    pltpu.sync_copy(dst_hbm.at[sid], dst_v)
    pltpu.sync_copy(params_hbm, params_v)

    zeros16 = jnp.zeros((16,), jnp.float32)

    def zb_body(j, _):
        for m in range(hd // 16):
            zbuf_v[j, pl.ds(m * 16, 16)] = zeros16
        return 0

    lax.fori_loop(0, CHUNK, zb_body, 0)

    for k in range(8):
        ck = sid * 8 + k

        @pl.when(ck < out_chunks)
        def _zero_chunk():
            pltpu.sync_copy(zbuf_v, acc_sp.at[pl.ds(ck * CHUNK, CHUNK)])

    @pl.when(sid == 0)
    def _zero_den():
        def zd_body(j, _):
            b_v[pl.ds(j * 16, 16)] = zeros16
            return 0

        lax.fori_loop(0, n // 16, zd_body, 0)
        pltpu.sync_copy(b_v.at[pl.ds(0, n)], den_sp)

    plsc.subcore_barrier()

    pltpu.sync_copy(b_hbm.at[pl.ds(base, ept)], b_v)

    params16 = params_v[...]
    c = params16[0]
    gmax = params16[1]

    def p1_body(j, _):
        b16 = b_v[pl.ds(j * 16, 16)]
        bc = b16 + c
        ea = jnp.where(bc > 0, bc, 0.01 * bc)
        b_v[pl.ds(j * 16, 16)] = jnp.exp(ea - gmax)
        return 0

    lax.fori_loop(0, ept // 16, p1_body, 0)

    zr = (zr0, zr1)
    ez = (ez0, ez1)
    gs = (gs0, gs1)
    es = (es0, es1)
    ss = (ss0, ss1)

    def drain_scat(i, p):
        pltpu.make_async_copy(zr[p], acc_sp.at[dst_v.at[i]], ss[p]).wait()

    def issue(i, p):
        pltpu.async_copy(zs_hbm.at[cid].at[src_v.at[i]], zr[p], gs[p])
        pltpu.async_copy(ezs_hbm.at[cid, pl.ds(base + i * CHUNK, CHUNK)],
                         ez[p], es[p])

    def process(i, p):
        pltpu.make_async_copy(zs_hbm.at[cid].at[src_v.at[i]], zr[p],
                              gs[p]).wait()
        pltpu.make_async_copy(ezs_hbm.at[cid, pl.ds(base, CHUNK)], ez[p],
                              es[p]).wait()
        pltpu.async_copy(b_v.at[pl.ds(i * CHUNK, CHUNK)],
                         den_sp.at[dst_v.at[i]], dsem, add=True)
        for q in range(CHUNK // 16):
            ev16 = b_v[pl.ds(i * CHUNK + q * 16, 16)]
            for r in range(16):
                j = q * 16 + r
                ev = ev16[r]
                for m in range(hd // 16):
                    sl = pl.ds(m * 16, 16)
                    zr[p][j, sl] = ev * (zr[p][j, sl] + ez[p][j, sl])
        pltpu.async_copy(zr[p], acc_sp.at[dst_v.at[i]], ss[p], add=True)

    issue(0, 0)

    def chunk_body(i2, _):
        i0 = i2 * 2

        @pl.when(i0 >= 1)
        def _d1():
            drain_scat(i0 - 1, 1)

        issue(i0 + 1, 1)
        process(i0, 0)
        process(i0 + 1, 1)
        drain_scat(i0, 0)

        @pl.when(i0 + 2 < nchunks)
        def _g0():
            issue(i0 + 2, 0)

        return 0

    lax.fori_loop(0, nchunks // 2, chunk_body, 0)
    drain_scat(nchunks - 1, 1)

    def den_drain(i, _):
        pltpu.make_async_copy(b_v.at[pl.ds(0, CHUNK)],
                              den_sp.at[dst_v.at[0]], dsem).wait()
        return 0

    lax.fori_loop(0, nchunks, den_drain, 0)

    plsc.subcore_barrier()

    for k in range(8):
        ck = sid * 8 + k

        @pl.when(ck < out_chunks)
        def _copy_chunk():
            pltpu.sync_copy(acc_sp.at[pl.ds(ck * CHUNK, CHUNK)], zbuf_v)
            pltpu.sync_copy(zbuf_v, accp_hbm.at[cid, pl.ds(ck * CHUNK, CHUNK)])

    @pl.when(sid == 0)
    def _copy_den():
        pltpu.sync_copy(den_sp, b_v.at[pl.ds(0, n)])
        pltpu.sync_copy(b_v.at[pl.ds(0, n)], denp_hbm.at[cid, 0])


def _phase_d(src3, dst3, b, params, zs, ezs, n, n_e, d):
    hd = d // 2
    ept = n_e // NS
    mesh = plsc.VectorSubcoreMesh(core_axis_name="c", subcore_axis_name="s")
    return pl.kernel(
        functools.partial(_kd_body, n, n_e, hd),
        compiler_params=pltpu.CompilerParams(needs_layout_passes=False,
                                             use_tc_tiling_on_sc=False),
        out_type=[
            jax.ShapeDtypeStruct((NC, n, hd), jnp.float32),
            jax.ShapeDtypeStruct((NC, 1, n), jnp.float32),
        ],
        mesh=mesh,
        scratch_types=[
            pltpu.VMEM((ept // CHUNK, CHUNK), jnp.int32),
            pltpu.VMEM((ept // CHUNK, CHUNK), jnp.int32),
            pltpu.VMEM((ept,), jnp.float32),
            pltpu.VMEM((16,), jnp.float32),
            pltpu.VMEM((CHUNK, hd), jnp.float32),
            pltpu.VMEM((CHUNK, hd), jnp.float32),
            pltpu.VMEM((CHUNK, hd), jnp.float32),
            pltpu.VMEM((CHUNK, hd), jnp.float32),
            pltpu.VMEM((CHUNK, hd), jnp.float32),
            pltpu.SemaphoreType.DMA,
            pltpu.SemaphoreType.DMA,
            pltpu.SemaphoreType.DMA,
            pltpu.SemaphoreType.DMA,
            pltpu.SemaphoreType.DMA,
            pltpu.SemaphoreType.DMA,
            pltpu.SemaphoreType.DMA,
            pltpu.VMEM_SHARED((n, hd), jnp.float32),
            pltpu.VMEM_SHARED((n,), jnp.float32),
        ],
    )(src3, dst3, b, params, zs, ezs)


def _ke_body(accp_ref, denp_ref, out_ref):
    acc = jnp.concatenate([accp_ref[0], accp_ref[1]], axis=1)
    db = denp_ref[0, 0, 0]
    out_ref[...] = acc / jnp.maximum(db, 1e-9)[:, None]


def _phase_e(accp, denp, n, d, nb):
    grid = n // nb
    denp0 = denp.reshape(NC, grid, 1, nb)
    return pl.pallas_call(
        _ke_body,
        grid=(grid,),
        in_specs=[
            pl.BlockSpec((NC, nb, d // 2), lambda i: (0, i, 0)),
            pl.BlockSpec((1, 1, 1, nb), lambda i: (0, i, 0, 0)),
        ],
        out_specs=pl.BlockSpec((nb, d), lambda i: (i, 0)),
        out_shape=jax.ShapeDtypeStruct((n, d), jnp.float32),
    )(accp, denp0)


def kernel(h, edge_index, e, W_fc, W_attn, W_eatt, W_e2n, W_m1, W_m2):
    n, _ = h.shape
    d = W_fc.shape[1]
    n_e = edge_index.shape[1]
    ed = e.shape[1]
    ept = n_e // NS

    wa = W_attn[:, 0]
    wst = jnp.stack([wa[0:d], wa[3 * d:4 * d]], axis=1)
    w7 = W_attn[6 * d:, :]

    src = edge_index[0]
    dst = edge_index[1]
    src3 = src.reshape(NS, ept // CHUNK, CHUNK)
    dst3 = dst.reshape(NS, ept // CHUNK, CHUNK)

    zs, st3 = _phase_a(h, W_fc, wst, n, d, 1000)
    st = st3.transpose(1, 0, 2).reshape(2, 1, n)
    ezs, u3 = _phase_b1(e, W_eatt, W_e2n, w7, n_e, ed, d, 2000)
    u = u3.reshape(n_e)

    b, bmaxp, cnts, cntd = _phase_b2(st, u, src, dst, n, n_e)
    params = _phase_c(cnts, cntd, zs, bmaxp, W_m1, W_m2, W_attn, n, d, n_e)
    accp, denp = _phase_d(src3, dst3, b, params, zs, ezs, n, n_e, d)
    return _phase_e(accp, denp, n, d, 1000)


# speedup vs baseline: 1.0463x; 1.0463x over previous
import functools

import jax
import jax.numpy as jnp
from jax import lax
from jax.experimental import pallas as pl
from jax.experimental.pallas import tpu as pltpu
from jax.experimental.pallas import tpu_sc as plsc

NC = 2
NS = 16
NW = NC * NS

CHUNK = 80
NEG_INF = -3.0e38


def _ka_body(h_ref, wfc_ref, wst_ref, zs_ref, st_ref):
    zb = jnp.dot(h_ref[...], wfc_ref[...], preferred_element_type=jnp.float32)
    hd = zb.shape[1] // 2
    zs_ref[...] = jnp.stack([zb[:, :hd], zb[:, hd:]], axis=0)
    st = jnp.dot(zb, wst_ref[...], preferred_element_type=jnp.float32)
    st_ref[...] = st.T[None]


def _phase_a(h, W_fc, wst, n, d, nb):
    grid = n // nb
    return pl.pallas_call(
        _ka_body,
        grid=(grid,),
        in_specs=[
            pl.BlockSpec((nb, d), lambda i: (i, 0)),
            pl.BlockSpec((d, d), lambda i: (0, 0)),
            pl.BlockSpec((d, 2), lambda i: (0, 0)),
        ],
        out_specs=[
            pl.BlockSpec((2, nb, d // 2), lambda i: (0, i, 0)),
            pl.BlockSpec((1, 2, nb), lambda i: (i, 0, 0)),
        ],
        out_shape=[
            jax.ShapeDtypeStruct((2, n, d // 2), jnp.float32),
            jax.ShapeDtypeStruct((grid, 2, nb), jnp.float32),
        ],
    )(h, W_fc, wst)


def _kb1_body(e_ref, weatt_ref, we2n_ref, w7_ref, ezs_ref, u_ref):
    v2 = jnp.dot(weatt_ref[...], we2n_ref[...], preferred_element_type=jnp.float32)
    v7 = jnp.dot(weatt_ref[...], w7_ref[...], preferred_element_type=jnp.float32)
    eb = e_ref[...]
    ez = jnp.dot(eb, v2, preferred_element_type=jnp.float32)
    hd = ez.shape[1] // 2
    ezs_ref[...] = jnp.stack([ez[:, :hd], ez[:, hd:]], axis=0)
    u = jnp.dot(eb, v7, preferred_element_type=jnp.float32)
    u_ref[...] = u.reshape(1, 1, -1)


def _phase_b1(e, W_eatt, W_e2n, w7, n_e, ed, d, eb):
    grid = n_e // eb
    return pl.pallas_call(
        _kb1_body,
        grid=(grid,),
        in_specs=[
            pl.BlockSpec((eb, ed), lambda i: (i, 0)),
            pl.BlockSpec((ed, ed), lambda i: (0, 0)),
            pl.BlockSpec((ed, d), lambda i: (0, 0)),
            pl.BlockSpec((ed, 1), lambda i: (0, 0)),
        ],
        out_specs=[
            pl.BlockSpec((2, eb, d // 2), lambda i: (0, i, 0)),
            pl.BlockSpec((1, 1, eb), lambda i: (i, 0, 0)),
        ],
        out_shape=[
            jax.ShapeDtypeStruct((2, n_e, d // 2), jnp.float32),
            jax.ShapeDtypeStruct((grid, 1, eb), jnp.float32),
        ],
    )(e, W_eatt, W_e2n, w7)


def _kb2_body(n, epw, st_hbm, u_hbm, src_hbm, dst_hbm,
              b_hbm, bmax_hbm, cnts_hbm, cntd_hbm,
              s_v, t_v, u_v, src_v, dst_v, b_v, cnt_s, cnt_d, bmax_v):
    cid = lax.axis_index("c")
    sid = lax.axis_index("s")
    wid = sid * NC + cid
    base = wid * epw

    pltpu.sync_copy(st_hbm.at[0, 0], s_v)
    pltpu.sync_copy(st_hbm.at[1, 0], t_v)
    pltpu.sync_copy(u_hbm.at[pl.ds(base, epw)], u_v)
    pltpu.sync_copy(src_hbm.at[pl.ds(base, epw)], src_v)
    pltpu.sync_copy(dst_hbm.at[pl.ds(base, epw)], dst_v)

    zeros16 = jnp.zeros((16,), jnp.float32)

    def zero_body(j, _):
        cnt_s[pl.ds(j * 16, 16)] = zeros16
        cnt_d[pl.ds(j * 16, 16)] = zeros16
        return 0

    lax.fori_loop(0, n // 16, zero_body, 0)

    onesf = jnp.ones((16,), jnp.float32)

    def body(j, bmax):
        sl = pl.ds(j * 16, 16)
        src16 = src_v[sl]
        dst16 = dst_v[sl]
        s16 = plsc.load_gather(s_v, [src16])
        t16 = plsc.load_gather(t_v, [dst16])
        b16 = s16 + t16 + u_v[sl]
        b_v[sl] = b16
        plsc.addupdate_scatter(cnt_s, [src16], onesf)
        plsc.addupdate_scatter(cnt_d, [dst16], onesf)
        return jnp.maximum(bmax, b16)

    bmax = lax.fori_loop(0, epw // 16, body, jnp.full((16,), NEG_INF, jnp.float32))
    bmax_v[...] = bmax

    pltpu.sync_copy(b_v, b_hbm.at[pl.ds(base, epw)])
    pltpu.sync_copy(bmax_v, bmax_hbm.at[wid, 0])
    pltpu.sync_copy(cnt_s, cnts_hbm.at[wid, 0])
    pltpu.sync_copy(cnt_d, cntd_hbm.at[wid, 0])


def _phase_b2(st, u, src, dst, n, n_e):
    epw = n_e // NW
    mesh = plsc.VectorSubcoreMesh(core_axis_name="c", subcore_axis_name="s")
    return pl.kernel(
        functools.partial(_kb2_body, n, epw),
        compiler_params=pltpu.CompilerParams(needs_layout_passes=False),
        out_type=[
            jax.ShapeDtypeStruct((n_e,), jnp.float32),
            jax.ShapeDtypeStruct((NW, 1, 16), jnp.float32),
            jax.ShapeDtypeStruct((NW, 1, n), jnp.float32),
            jax.ShapeDtypeStruct((NW, 1, n), jnp.float32),
        ],
        mesh=mesh,
        scratch_types=[
            pltpu.VMEM((n,), jnp.float32),
            pltpu.VMEM((n,), jnp.float32),
            pltpu.VMEM((epw,), jnp.float32),
            pltpu.VMEM((epw,), jnp.int32),
            pltpu.VMEM((epw,), jnp.int32),
            pltpu.VMEM((epw,), jnp.float32),
            pltpu.VMEM((n,), jnp.float32),
            pltpu.VMEM((n,), jnp.float32),
            pltpu.VMEM((16,), jnp.float32),
        ],
    )(st, u, src, dst)


def _kc_body(n_e, cnts_ref, cntd_ref, zs_ref, bmaxp_ref, wm1_ref, wm2_ref,
             wattn_ref, params_ref):
    cnt = jnp.stack([jnp.sum(cnts_ref[...], axis=(0, 1)),
                     jnp.sum(cntd_ref[...], axis=(0, 1))])
    z = jnp.concatenate([zs_ref[0], zs_ref[1]], axis=1)
    inv_e = 1.0 / n_e
    ex1 = jnp.dot(cnt, z, preferred_element_type=jnp.float32) * inv_e
    ex2 = jnp.dot(cnt, z * z, preferred_element_type=jnp.float32) * inv_e
    ex3 = jnp.dot(cnt, z * z * z, preferred_element_type=jnp.float32) * inv_e
    m2 = ex2 - ex1 * ex1
    m3 = ex3 - 3.0 * ex1 * ex2 + 2.0 * ex1 * ex1 * ex1
    sr2 = jnp.sign(m2) * jnp.sqrt(jnp.abs(m2))
    sr3 = jnp.sign(m3) * jnp.abs(m3) ** (1.0 / 3.0)
    a2 = jnp.dot(sr2, wm1_ref[...], preferred_element_type=jnp.float32)
    a3 = jnp.dot(sr3, wm2_ref[...], preferred_element_type=jnp.float32)
    wa = wattn_ref[...][:, 0]
    d = z.shape[1]
    c = (jnp.sum(a2[0] * wa[d:2 * d]) + jnp.sum(a3[0] * wa[2 * d:3 * d])
         + jnp.sum(a2[1] * wa[4 * d:5 * d]) + jnp.sum(a3[1] * wa[5 * d:6 * d]))
    gb = jnp.max(bmaxp_ref[...]) + c
    gmax = jnp.where(gb > 0, gb, 0.01 * gb)
    lane = lax.broadcasted_iota(jnp.int32, (16,), 0)
    params_ref[...] = jnp.where(lane == 0, c, jnp.where(lane == 1, gmax, 0.0))


def _phase_c(cnts, cntd, zs, bmaxp, W_m1, W_m2, W_attn, n, d, n_e):
    return pl.pallas_call(
        functools.partial(_kc_body, float(n_e)),
        grid=(1,),
        in_specs=[
            pl.BlockSpec((NW, 1, n), lambda i: (0, 0, 0)),
            pl.BlockSpec((NW, 1, n), lambda i: (0, 0, 0)),
            pl.BlockSpec((2, n, d // 2), lambda i: (0, 0, 0)),
            pl.BlockSpec((NW, 1, 16), lambda i: (0, 0, 0)),
            pl.BlockSpec((d, d), lambda i: (0, 0)),
            pl.BlockSpec((d, d), lambda i: (0, 0)),
            pl.BlockSpec((6 * d + 16, 1), lambda i: (0, 0)),
        ],
        out_specs=pl.BlockSpec((16,), lambda i: (0,)),
        out_shape=jax.ShapeDtypeStruct((16,), jnp.float32),
    )(cnts, cntd, zs, bmaxp, W_m1, W_m2, W_attn)


def _kd_body(n, n_e, hd, src_hbm, dst_hbm, b_hbm, params_hbm, zs_hbm, ezs_hbm,
             accp_hbm, denp_hbm,
             src_v, dst_v, b_v, params_v, zr0, zr1, ez0, ez1, zbuf_v,
             gs0, gs1, es0, es1, ss0, ss1, dsem,
             acc_sp, den_sp):
    cid = lax.axis_index("c")
    sid = lax.axis_index("s")
    ept = n_e // NS
    base = sid * ept
    nchunks = ept // CHUNK
    out_chunks = n // CHUNK

    pltpu.sync_copy(src_hbm.at[sid], src_v)
    pltpu.sync_copy(dst_hbm.at[sid], dst_v)
    pltpu.sync_copy(params_hbm, params_v)

    zeros16 = jnp.zeros((16,), jnp.float32)

    def zb_body(j, _):
        for m in range(hd // 16):
            zbuf_v[j, pl.ds(m * 16, 16)] = zeros16
        return 0

    lax.fori_loop(0, CHUNK, zb_body, 0)

    for k in range(8):
        ck = sid * 8 + k

        @pl.when(ck < out_chunks)
        def _zero_chunk():
            pltpu.sync_copy(zbuf_v, acc_sp.at[pl.ds(ck * CHUNK, CHUNK)])

    @pl.when(sid == 0)
    def _zero_den():
        def zd_body(j, _):
            b_v[pl.ds(j * 16, 16)] = zeros16
            return 0

        lax.fori_loop(0, n // 16, zd_body, 0)
        pltpu.sync_copy(b_v.at[pl.ds(0, n)], den_sp)

    plsc.subcore_barrier()

    pltpu.sync_copy(b_hbm.at[pl.ds(base, ept)], b_v)

    params16 = params_v[...]
    c = params16[0]
    gmax = params16[1]

    def p1_body(j, _):
        b16 = b_v[pl.ds(j * 16, 16)]
        bc = b16 + c
        ea = jnp.where(bc > 0, bc, 0.01 * bc)
        b_v[pl.ds(j * 16, 16)] = jnp.exp(ea - gmax)
        return 0

    lax.fori_loop(0, ept // 16, p1_body, 0)

    zr = (zr0, zr1)
    ez = (ez0, ez1)
    gs = (gs0, gs1)
    es = (es0, es1)
    ss = (ss0, ss1)

    def drain_scat(i, p):
        pltpu.make_async_copy(zr[p], acc_sp.at[dst_v.at[i]], ss[p]).wait()

    def issue(i, p):
        pltpu.async_copy(zs_hbm.at[cid].at[src_v.at[i]], zr[p], gs[p])
        pltpu.async_copy(ezs_hbm.at[cid, pl.ds(base + i * CHUNK, CHUNK)],
                         ez[p], es[p])

    def process(i, p):
        pltpu.make_async_copy(zs_hbm.at[cid].at[src_v.at[i]], zr[p],
                              gs[p]).wait()
        pltpu.make_async_copy(ezs_hbm.at[cid, pl.ds(base, CHUNK)], ez[p],
                              es[p]).wait()
        pltpu.async_copy(b_v.at[pl.ds(i * CHUNK, CHUNK)],
                         den_sp.at[dst_v.at[i]], dsem, add=True)
        for q in range(CHUNK // 16):
            ev16 = b_v[pl.ds(i * CHUNK + q * 16, 16)]
            for r in range(16):
                j = q * 16 + r
                ev = ev16[r]
                for m in range(hd // 16):
                    sl = pl.ds(m * 16, 16)
                    zr[p][j, sl] = ev * (zr[p][j, sl] + ez[p][j, sl])
        pltpu.sync_copy(zr[p], acc_sp.at[dst_v.at[i]], add=True)

    issue(0, 0)

    def chunk_body(i2, _):
        i0 = i2 * 2
        issue(i0 + 1, 1)
        process(i0, 0)

        @pl.when(i0 + 2 < nchunks)
        def _g0():
            issue(i0 + 2, 0)

        process(i0 + 1, 1)
        return 0

    lax.fori_loop(0, nchunks // 2, chunk_body, 0)

    def den_drain(i, _):
        pltpu.make_async_copy(b_v.at[pl.ds(0, CHUNK)],
                              den_sp.at[dst_v.at[0]], dsem).wait()
        return 0

    lax.fori_loop(0, nchunks, den_drain, 0)

    plsc.subcore_barrier()

    for k in range(8):
        ck = sid * 8 + k

        @pl.when(ck < out_chunks)
        def _copy_chunk():
            pltpu.sync_copy(acc_sp.at[pl.ds(ck * CHUNK, CHUNK)], zbuf_v)
            pltpu.sync_copy(zbuf_v, accp_hbm.at[cid, pl.ds(ck * CHUNK, CHUNK)])

    @pl.when(sid == 0)
    def _copy_den():
        pltpu.sync_copy(den_sp, b_v.at[pl.ds(0, n)])
        pltpu.sync_copy(b_v.at[pl.ds(0, n)], denp_hbm.at[cid, 0])


def _phase_d(src3, dst3, b, params, zs, ezs, n, n_e, d):
    hd = d // 2
    ept = n_e // NS
    mesh = plsc.VectorSubcoreMesh(core_axis_name="c", subcore_axis_name="s")
    return pl.kernel(
        functools.partial(_kd_body, n, n_e, hd),
        compiler_params=pltpu.CompilerParams(needs_layout_passes=False,
                                             use_tc_tiling_on_sc=False),
        out_type=[
            jax.ShapeDtypeStruct((NC, n, hd), jnp.float32),
            jax.ShapeDtypeStruct((NC, 1, n), jnp.float32),
        ],
        mesh=mesh,
        scratch_types=[
            pltpu.VMEM((ept // CHUNK, CHUNK), jnp.int32),
            pltpu.VMEM((ept // CHUNK, CHUNK), jnp.int32),
            pltpu.VMEM((ept,), jnp.float32),
            pltpu.VMEM((16,), jnp.float32),
            pltpu.VMEM((CHUNK, hd), jnp.float32),
            pltpu.VMEM((CHUNK, hd), jnp.float32),
            pltpu.VMEM((CHUNK, hd), jnp.float32),
            pltpu.VMEM((CHUNK, hd), jnp.float32),
            pltpu.VMEM((CHUNK, hd), jnp.float32),
            pltpu.SemaphoreType.DMA,
            pltpu.SemaphoreType.DMA,
            pltpu.SemaphoreType.DMA,
            pltpu.SemaphoreType.DMA,
            pltpu.SemaphoreType.DMA,
            pltpu.SemaphoreType.DMA,
            pltpu.SemaphoreType.DMA,
            pltpu.VMEM_SHARED((n, hd), jnp.float32),
            pltpu.VMEM_SHARED((n,), jnp.float32),
        ],
    )(src3, dst3, b, params, zs, ezs)


def _ke_body(accp_ref, denp_ref, out_ref):
    acc = jnp.concatenate([accp_ref[0], accp_ref[1]], axis=1)
    db = denp_ref[0, 0, 0]
    out_ref[...] = acc / jnp.maximum(db, 1e-9)[:, None]


def _phase_e(accp, denp, n, d, nb):
    grid = n // nb
    denp0 = denp.reshape(NC, grid, 1, nb)
    return pl.pallas_call(
        _ke_body,
        grid=(grid,),
        in_specs=[
            pl.BlockSpec((NC, nb, d // 2), lambda i: (0, i, 0)),
            pl.BlockSpec((1, 1, 1, nb), lambda i: (0, i, 0, 0)),
        ],
        out_specs=pl.BlockSpec((nb, d), lambda i: (i, 0)),
        out_shape=jax.ShapeDtypeStruct((n, d), jnp.float32),
    )(accp, denp0)


def kernel(h, edge_index, e, W_fc, W_attn, W_eatt, W_e2n, W_m1, W_m2):
    n, _ = h.shape
    d = W_fc.shape[1]
    n_e = edge_index.shape[1]
    ed = e.shape[1]
    ept = n_e // NS

    wa = W_attn[:, 0]
    wst = jnp.stack([wa[0:d], wa[3 * d:4 * d]], axis=1)
    w7 = W_attn[6 * d:, :]

    src = edge_index[0]
    dst = edge_index[1]
    src3 = src.reshape(NS, ept // CHUNK, CHUNK)
    dst3 = dst.reshape(NS, ept // CHUNK, CHUNK)

    zs, st3 = _phase_a(h, W_fc, wst, n, d, 1000)
    st = st3.transpose(1, 0, 2).reshape(2, 1, n)
    ezs, u3 = _phase_b1(e, W_eatt, W_e2n, w7, n_e, ed, d, 2000)
    u = u3.reshape(n_e)

    b, bmaxp, cnts, cntd = _phase_b2(st, u, src, dst, n, n_e)
    params = _phase_c(cnts, cntd, zs, bmaxp, W_m1, W_m2, W_attn, n, d, n_e)
    accp, denp = _phase_d(src3, dst3, b, params, zs, ezs, n, n_e, d)
    return _phase_e(accp, denp, n, d, 1000)
